# Initial kernel scaffold; baseline (speedup 1.0000x reference)
import jax
import jax.numpy as jnp
from jax.experimental import pallas as pl


def kernel(input_tensor, L_rows, L_cols, L_vals, kernel):
    raise NotImplementedError("write your pallas kernel here")


# speedup vs baseline: 3.2979x; 3.2979x over previous
import functools

import jax
import jax.numpy as jnp
from jax import lax
from jax.experimental import pallas as pl
from jax.experimental.pallas import tpu as pltpu
from jax.experimental.pallas import tpu_sc as plsc

_NC = 2
_NS = 16
_LANES = 16
_NW = _NC * _NS
_CHUNK = 80
_FLUSH_ROWS = 80


def _spmm_partials(x, rows, cols, vals):
  M, F = x.shape
  E = rows.shape[0]
  e_per_w = E // _NW
  n_chunks = e_per_w // _CHUNK
  tile_rows = -(-(M // _NS) // _FLUSH_ROWS) * _FLUSH_ROWS

  mesh = plsc.VectorSubcoreMesh(
      core_axis_name="c", subcore_axis_name="s",
      num_cores=_NC, num_subcores=_NS)

  @functools.partial(
      pl.kernel,
      out_type=jax.ShapeDtypeStruct((_NC, M, F), jnp.float32),
      mesh=mesh,
      compiler_params=pltpu.CompilerParams(needs_layout_passes=False),
      scratch_types=[
          pltpu.VMEM((_CHUNK,), jnp.int32),
          pltpu.VMEM((_CHUNK,), jnp.int32),
          pltpu.VMEM((_CHUNK,), jnp.float32),
          pltpu.VMEM((_CHUNK, F), jnp.float32),
          pltpu.VMEM_SHARED((M, F), jnp.float32),
      ],
  )
  def run(x_hbm, rows_hbm, cols_hbm, vals_hbm, out_hbm,
          rowbuf, colbuf, valbuf, xbuf, accum):
    cid = lax.axis_index("c")
    sid = lax.axis_index("s")
    wid = cid * _NS + sid

    zero16 = jnp.zeros((_LANES,), jnp.float32)

    @pl.loop(0, _CHUNK)
    def _(r):
      for j in range(F // _LANES):
        xbuf[r, pl.ds(j * _LANES, _LANES)] = zero16

    start = sid * tile_rows
    count = jnp.minimum(tile_rows, M - start)

    @pl.loop(0, count // _FLUSH_ROWS)
    def _(i):
      pltpu.sync_copy(xbuf.at[pl.ds(0, _FLUSH_ROWS)],
                      accum.at[pl.ds(start + i * _FLUSH_ROWS, _FLUSH_ROWS)])

    plsc.subcore_barrier()

    base = wid * e_per_w

    @pl.loop(0, n_chunks)
    def _(i):
      off = base + i * _CHUNK
      pltpu.sync_copy(rows_hbm.at[pl.ds(off, _CHUNK)], rowbuf)
      pltpu.sync_copy(cols_hbm.at[pl.ds(off, _CHUNK)], colbuf)
      pltpu.sync_copy(vals_hbm.at[pl.ds(off, _CHUNK)], valbuf)
      pltpu.sync_copy(x_hbm.at[colbuf], xbuf)

      @pl.loop(0, _CHUNK)
      def _(e):
        idx = jnp.full((_LANES,), e, jnp.int32)
        v = plsc.load_gather(valbuf, [idx])
        for j in range(F // _LANES):
          sl = pl.ds(j * _LANES, _LANES)
          xbuf[e, sl] = xbuf[e, sl] * v

      pltpu.sync_copy(xbuf, accum.at[rowbuf], add=True)

    plsc.subcore_barrier()

    @pl.loop(0, count // _FLUSH_ROWS)
    def _(i):
      r = start + i * _FLUSH_ROWS
      pltpu.sync_copy(accum.at[pl.ds(r, _FLUSH_ROWS)], xbuf)
      pltpu.sync_copy(xbuf, out_hbm.at[cid, pl.ds(r, _FLUSH_ROWS)])

  return run(x, rows, cols, vals)


def _combine(p, x_prev, a, b):
  M, F = x_prev.shape
  blk = 1000

  def body(p0_ref, p1_ref, xp_ref, o_ref):
    o_ref[...] = a * (p0_ref[...] + p1_ref[...]) + b * xp_ref[...]

  spec = pl.BlockSpec((blk, F), lambda i: (i, 0))
  return pl.pallas_call(
      body,
      grid=(M // blk,),
      in_specs=[spec, spec, spec],
      out_specs=spec,
      out_shape=jax.ShapeDtypeStruct((M, F), jnp.float32),
  )(p[0], p[1], x_prev)


def _cheb_matmul(xs, w):
  M, F = xs[0].shape
  K, _, FOUT = w.shape
  blk = 1000

  def body(*refs):
    x_refs, w_ref, o_ref = refs[:K], refs[K], refs[K + 1]
    acc = jnp.zeros((blk, FOUT), jnp.float32)
    for k in range(K):
      acc += jnp.dot(x_refs[k][...], w_ref[k],
                     preferred_element_type=jnp.float32)
    o_ref[...] = acc

  xspec = pl.BlockSpec((blk, F), lambda i: (i, 0))
  wspec = pl.BlockSpec((K, F, FOUT), lambda i: (0, 0, 0))
  return pl.pallas_call(
      body,
      grid=(M // blk,),
      in_specs=[xspec] * K + [wspec],
      out_specs=pl.BlockSpec((blk, FOUT), lambda i: (i, 0)),
      out_shape=jax.ShapeDtypeStruct((M, FOUT), jnp.float32),
  )(*xs, w)


def kernel(input_tensor, L_rows, L_cols, L_vals, kernel):
  B, M, F = input_tensor.shape
  KF, FOUT = kernel.shape
  K = KF // F

  x0 = input_tensor.reshape(M, F)
  w = kernel.reshape(F, K, FOUT).transpose(1, 0, 2)

  xs = [x0]
  if K > 1:
    p = _spmm_partials(x0, L_rows, L_cols, L_vals)
    xs.append(_combine(p, x0, 1.0, 0.0))
  for _ in range(2, K):
    p = _spmm_partials(xs[-1], L_rows, L_cols, L_vals)
    xs.append(_combine(p, xs[-2], 2.0, -1.0))

  out = _cheb_matmul(xs, w)
  return out.reshape(B, M, FOUT)
